# Initial kernel scaffold; baseline (speedup 1.0000x reference)
import jax
import jax.numpy as jnp
from jax.experimental import pallas as pl


def kernel(x, pos, batch, W1, b1, W2, b2, W3, b3):
    raise NotImplementedError("write your pallas kernel here")


# speedup vs baseline: 1.0196x; 1.0196x over previous
import jax
import jax.numpy as jnp
import numpy as np
from jax.experimental import pallas as pl
from jax.experimental.pallas import tpu as pltpu

_B = 16
_NP = 4096
_S = 512
_K = 64
_D = 64
_R = 0.2


def _fps_idx(pos_b):
    d = jnp.sum((pos_b - pos_b[0]) ** 2, axis=-1)
    idxs = jnp.zeros((_S,), dtype=jnp.int32)

    def body(i, state):
        d, idxs = state
        nxt = jnp.argmax(d).astype(jnp.int32)
        idxs = idxs.at[i].set(nxt)
        d = jnp.minimum(d, jnp.sum((pos_b - pos_b[nxt]) ** 2, axis=-1))
        return (d, idxs)

    d, idxs = jax.lax.fori_loop(1, _S, body, (d, idxs))
    return idxs


def _mlp_body(g_ref, w1_ref, b1_ref, w2_ref, b2_ref, w3_ref, b3_ref, o_ref):
    g = g_ref[...]
    h = jnp.maximum(
        jnp.dot(g, w1_ref[...], preferred_element_type=jnp.float32) + b1_ref[...], 0.0)
    h = jnp.maximum(
        jnp.dot(h, w2_ref[...], preferred_element_type=jnp.float32) + b2_ref[...], 0.0)
    h = jnp.maximum(
        jnp.dot(h, w3_ref[...], preferred_element_type=jnp.float32) + b3_ref[...], 0.0)
    for _ in range(6):
        n = h.shape[0] // 2
        hr = h.reshape(n, 2, 128)
        h = jnp.maximum(hr[:, 0, :], hr[:, 1, :])
    o_ref[...] = h


def _mlp_max(g, W1, b1, W2, b2, W3, b3):
    npairs = g.shape[0]
    QB = 64
    rows = QB * _K
    grid = (npairs // rows,)
    return pl.pallas_call(
        _mlp_body,
        grid=grid,
        in_specs=[
            pl.BlockSpec((rows, 67), lambda i: (i, 0)),
            pl.BlockSpec((67, 64), lambda i: (0, 0)),
            pl.BlockSpec((1, 64), lambda i: (0, 0)),
            pl.BlockSpec((64, 64), lambda i: (0, 0)),
            pl.BlockSpec((1, 64), lambda i: (0, 0)),
            pl.BlockSpec((64, 128), lambda i: (0, 0)),
            pl.BlockSpec((1, 128), lambda i: (0, 0)),
        ],
        out_specs=pl.BlockSpec((QB, 128), lambda i: (i, 0)),
        out_shape=jax.ShapeDtypeStruct((npairs // _K, 128), jnp.float32),
    )(g, W1, b1.reshape(1, 64), W2, b2.reshape(1, 64), W3, b3.reshape(1, 128))


def kernel(x, pos, batch, W1, b1, W2, b2, W3, b3):
    pos_b = pos.reshape(_B, _NP, 3)
    x_b = x.reshape(_B, _NP, _D)
    idx_local = jax.vmap(_fps_idx)(pos_b)
    q = jax.vmap(lambda pb, il: pb[il])(pos_b, idx_local)
    d2 = jnp.sum((q[:, :, None, :] - pos_b[:, None, :, :]) ** 2, axis=-1)
    mask = d2 <= _R * _R
    score = jnp.where(mask, -d2, -jnp.inf)
    vals, nbr = jax.lax.top_k(score, _K)
    valid = vals > -jnp.inf
    self_idx = idx_local[:, :, None]
    nbr = jnp.where(valid, nbr, self_idx)
    x_nbr = jax.vmap(lambda xb, nb: xb[nb])(x_b, nbr)
    pos_nbr = jax.vmap(lambda pb, nb: pb[nb])(pos_b, nbr)
    rel = pos_nbr - q[:, :, None, :]
    g = jnp.concatenate([x_nbr, rel], axis=-1).reshape(_B * _S * _K, _D + 3)
    out = _mlp_max(g, W1, b1, W2, b2, W3, b3)
    idx_global = (idx_local + jnp.arange(_B, dtype=jnp.int32)[:, None] * _NP).reshape(-1)
    x_out = out
    pos_out = q.reshape(_B * _S, 3)
    batch_out = batch[idx_global]
    return (x_out, pos_out, batch_out)


# speedup vs baseline: 1.3108x; 1.2856x over previous
import jax
import jax.numpy as jnp
import numpy as np
from jax.experimental import pallas as pl
from jax.experimental.pallas import tpu as pltpu

_B = 16
_NP = 4096
_S = 512
_K = 64
_D = 64
_R = 0.2


_SL = 32


def _fps_body(px_ref, py_ref, pz_ref, idx_ref, qx_ref, qy_ref, qz_ref, d_ref):
    px = px_ref[...]
    py = py_ref[...]
    pz = pz_ref[...]
    li = (jax.lax.broadcasted_iota(jnp.int32, (1, _SL, 128), 1) * 128
          + jax.lax.broadcasted_iota(jnp.int32, (1, _SL, 128), 2))
    li_s = (jax.lax.broadcasted_iota(jnp.int32, (1, 4, 128), 1) * 128
            + jax.lax.broadcasted_iota(jnp.int32, (1, 4, 128), 2))

    def point_coords(nxt):
        sel = li == nxt
        sx = jnp.sum(jnp.where(sel, px, 0.0), axis=(1, 2), keepdims=True)
        sy = jnp.sum(jnp.where(sel, py, 0.0), axis=(1, 2), keepdims=True)
        sz = jnp.sum(jnp.where(sel, pz, 0.0), axis=(1, 2), keepdims=True)
        return sx, sy, sz

    zero = jnp.zeros((_B, 1, 1), jnp.int32)
    sx, sy, sz = point_coords(zero)
    d_ref[...] = (px - sx) ** 2 + (py - sy) ** 2 + (pz - sz) ** 2
    idx_ref[...] = jnp.zeros((_B, 4, 128), jnp.int32)
    qx_ref[...] = jnp.where(li_s == 0, sx, 0.0)
    qy_ref[...] = jnp.where(li_s == 0, sy, 0.0)
    qz_ref[...] = jnp.where(li_s == 0, sz, 0.0)

    def body(i, _):
        d = d_ref[...]
        m = jnp.max(d, axis=(1, 2), keepdims=True)
        nxt = jnp.min(jnp.where(d == m, li, jnp.int32(1 << 30)),
                      axis=(1, 2), keepdims=True)
        sx, sy, sz = point_coords(nxt)
        dn = (px - sx) ** 2 + (py - sy) ** 2 + (pz - sz) ** 2
        d_ref[...] = jnp.minimum(d, dn)
        at_i = li_s == i
        idx_ref[...] = jnp.where(at_i, nxt, idx_ref[...])
        qx_ref[...] = jnp.where(at_i, sx, qx_ref[...])
        qy_ref[...] = jnp.where(at_i, sy, qy_ref[...])
        qz_ref[...] = jnp.where(at_i, sz, qz_ref[...])
        return 0

    jax.lax.fori_loop(1, _S, body, 0)


def _fps_all(pos_b):
    px = pos_b[:, :, 0].reshape(_B, _SL, 128)
    py = pos_b[:, :, 1].reshape(_B, _SL, 128)
    pz = pos_b[:, :, 2].reshape(_B, _SL, 128)
    out_shapes = (
        jax.ShapeDtypeStruct((_B, 4, 128), jnp.int32),
        jax.ShapeDtypeStruct((_B, 4, 128), jnp.float32),
        jax.ShapeDtypeStruct((_B, 4, 128), jnp.float32),
        jax.ShapeDtypeStruct((_B, 4, 128), jnp.float32),
    )
    idxs, qx, qy, qz = pl.pallas_call(
        _fps_body,
        out_shape=out_shapes,
        scratch_shapes=[pltpu.VMEM((_B, _SL, 128), jnp.float32)],
    )(px, py, pz)
    idx_local = idxs.reshape(_B, _S)
    q = jnp.stack([qx.reshape(_B, _S), qy.reshape(_B, _S), qz.reshape(_B, _S)],
                  axis=-1)
    return idx_local, q


def _mlp_body(g_ref, w1_ref, b1_ref, w2_ref, b2_ref, w3_ref, b3_ref, o_ref):
    g = g_ref[...]
    h = jnp.maximum(
        jnp.dot(g, w1_ref[...], preferred_element_type=jnp.float32) + b1_ref[...], 0.0)
    h = jnp.maximum(
        jnp.dot(h, w2_ref[...], preferred_element_type=jnp.float32) + b2_ref[...], 0.0)
    h = jnp.maximum(
        jnp.dot(h, w3_ref[...], preferred_element_type=jnp.float32) + b3_ref[...], 0.0)
    for _ in range(6):
        n = h.shape[0] // 2
        hr = h.reshape(n, 2, 128)
        h = jnp.maximum(hr[:, 0, :], hr[:, 1, :])
    o_ref[...] = h


def _mlp_max(g, W1, b1, W2, b2, W3, b3):
    npairs = g.shape[0]
    QB = 64
    rows = QB * _K
    grid = (npairs // rows,)
    return pl.pallas_call(
        _mlp_body,
        grid=grid,
        in_specs=[
            pl.BlockSpec((rows, 67), lambda i: (i, 0)),
            pl.BlockSpec((67, 64), lambda i: (0, 0)),
            pl.BlockSpec((1, 64), lambda i: (0, 0)),
            pl.BlockSpec((64, 64), lambda i: (0, 0)),
            pl.BlockSpec((1, 64), lambda i: (0, 0)),
            pl.BlockSpec((64, 128), lambda i: (0, 0)),
            pl.BlockSpec((1, 128), lambda i: (0, 0)),
        ],
        out_specs=pl.BlockSpec((QB, 128), lambda i: (i, 0)),
        out_shape=jax.ShapeDtypeStruct((npairs // _K, 128), jnp.float32),
    )(g, W1, b1.reshape(1, 64), W2, b2.reshape(1, 64), W3, b3.reshape(1, 128))


def kernel(x, pos, batch, W1, b1, W2, b2, W3, b3):
    pos_b = pos.reshape(_B, _NP, 3)
    x_b = x.reshape(_B, _NP, _D)
    idx_local, q = _fps_all(pos_b)
    d2 = jnp.sum((q[:, :, None, :] - pos_b[:, None, :, :]) ** 2, axis=-1)
    mask = d2 <= _R * _R
    score = jnp.where(mask, -d2, -jnp.inf)
    vals, nbr = jax.lax.top_k(score, _K)
    valid = vals > -jnp.inf
    self_idx = idx_local[:, :, None]
    nbr = jnp.where(valid, nbr, self_idx)
    x_nbr = jax.vmap(lambda xb, nb: xb[nb])(x_b, nbr)
    pos_nbr = jax.vmap(lambda pb, nb: pb[nb])(pos_b, nbr)
    rel = pos_nbr - q[:, :, None, :]
    g = jnp.concatenate([x_nbr, rel], axis=-1).reshape(_B * _S * _K, _D + 3)
    out = _mlp_max(g, W1, b1, W2, b2, W3, b3)
    idx_global = (idx_local + jnp.arange(_B, dtype=jnp.int32)[:, None] * _NP).reshape(-1)
    x_out = out
    pos_out = q.reshape(_B * _S, 3)
    batch_out = batch[idx_global]
    return (x_out, pos_out, batch_out)


# speedup vs baseline: 14.6784x; 11.1981x over previous
import dataclasses
import functools

import jax
import jax.numpy as jnp
import numpy as np
from jax import lax
from jax.experimental import pallas as pl
from jax.experimental.pallas import tpu as pltpu
from jax.experimental.pallas import tpu_sc as plsc

_B = 16
_NP = 4096
_S = 512
_K = 64
_D = 64
_R = 0.2


_SL = 32


def _fps_body(px_ref, py_ref, pz_ref, idx_ref, qx_ref, qy_ref, qz_ref, d_ref):
    px = px_ref[...]
    py = py_ref[...]
    pz = pz_ref[...]
    li = (jax.lax.broadcasted_iota(jnp.int32, (1, _SL, 128), 1) * 128
          + jax.lax.broadcasted_iota(jnp.int32, (1, _SL, 128), 2))
    li_s = (jax.lax.broadcasted_iota(jnp.int32, (1, 4, 128), 1) * 128
            + jax.lax.broadcasted_iota(jnp.int32, (1, 4, 128), 2))

    def point_coords(nxt):
        sel = li == nxt
        sx = jnp.sum(jnp.where(sel, px, 0.0), axis=(1, 2), keepdims=True)
        sy = jnp.sum(jnp.where(sel, py, 0.0), axis=(1, 2), keepdims=True)
        sz = jnp.sum(jnp.where(sel, pz, 0.0), axis=(1, 2), keepdims=True)
        return sx, sy, sz

    zero = jnp.zeros((_B, 1, 1), jnp.int32)
    sx, sy, sz = point_coords(zero)
    d_ref[...] = (px - sx) ** 2 + (py - sy) ** 2 + (pz - sz) ** 2
    idx_ref[...] = jnp.zeros((_B, 4, 128), jnp.int32)
    qx_ref[...] = jnp.where(li_s == 0, sx, 0.0)
    qy_ref[...] = jnp.where(li_s == 0, sy, 0.0)
    qz_ref[...] = jnp.where(li_s == 0, sz, 0.0)

    def body(i, _):
        d = d_ref[...]
        m = jnp.max(d, axis=(1, 2), keepdims=True)
        nxt = jnp.min(jnp.where(d == m, li, jnp.int32(1 << 30)),
                      axis=(1, 2), keepdims=True)
        sx, sy, sz = point_coords(nxt)
        dn = (px - sx) ** 2 + (py - sy) ** 2 + (pz - sz) ** 2
        d_ref[...] = jnp.minimum(d, dn)
        at_i = li_s == i
        idx_ref[...] = jnp.where(at_i, nxt, idx_ref[...])
        qx_ref[...] = jnp.where(at_i, sx, qx_ref[...])
        qy_ref[...] = jnp.where(at_i, sy, qy_ref[...])
        qz_ref[...] = jnp.where(at_i, sz, qz_ref[...])
        return 0

    jax.lax.fori_loop(1, _S, body, 0)


def _fps_all(pos_b):
    px = pos_b[:, :, 0].reshape(_B, _SL, 128)
    py = pos_b[:, :, 1].reshape(_B, _SL, 128)
    pz = pos_b[:, :, 2].reshape(_B, _SL, 128)
    out_shapes = (
        jax.ShapeDtypeStruct((_B, 4, 128), jnp.int32),
        jax.ShapeDtypeStruct((_B, 4, 128), jnp.float32),
        jax.ShapeDtypeStruct((_B, 4, 128), jnp.float32),
        jax.ShapeDtypeStruct((_B, 4, 128), jnp.float32),
    )
    idxs, qx, qy, qz = pl.pallas_call(
        _fps_body,
        out_shape=out_shapes,
        scratch_shapes=[pltpu.VMEM((_B, _SL, 128), jnp.float32)],
    )(px, py, pz)
    idx_local = idxs.reshape(_B, _S)
    q = jnp.stack([qx.reshape(_B, _S), qy.reshape(_B, _S), qz.reshape(_B, _S)],
                  axis=-1)
    return idx_local, q


_R2F = np.float32(_R * _R)
_R2BITS = int(np.float32(_R * _R).view(np.int32))


def _ballq_body(pos_ref, q_ref, d2_ref, tau_ref):
    px = pos_ref[0, 0:1, :]
    py = pos_ref[0, 1:2, :]
    pz = pos_ref[0, 2:3, :]
    qx = q_ref[0, :, 0:1]
    qy = q_ref[0, :, 1:2]
    qz = q_ref[0, :, 2:3]
    d2 = (qx - px) ** 2 + (qy - py) ** 2 + (qz - pz) ** 2
    d2_ref[0] = d2
    d2b = jax.lax.bitcast_convert_type(d2, jnp.int32)
    in_r = d2 <= _R2F
    d2m = jnp.where(in_r, d2b, jnp.int32(0x7F7FFFFF))
    cnt0 = jnp.sum(in_r.astype(jnp.int32), axis=1, keepdims=True)
    g = jnp.minimum(cnt0, _K)

    def body(_, state):
        lo, hi = state
        mid = (lo + hi) >> 1
        cnt = jnp.sum((d2m <= mid).astype(jnp.int32), axis=1, keepdims=True)
        ge = cnt >= g
        return (jnp.where(ge, lo, mid + 1), jnp.where(ge, mid, hi))

    lo = jnp.zeros((_S, 1), jnp.int32)
    hi = jnp.full((_S, 1), jnp.int32(_R2BITS))
    lo, hi = jax.lax.fori_loop(0, 30, body, (lo, hi))
    tau_ref[0] = jax.lax.bitcast_convert_type(hi, jnp.float32)


def _ballq(pos_b, q):
    pos_t = pos_b.transpose(0, 2, 1)
    d2, tau = pl.pallas_call(
        _ballq_body,
        grid=(_B,),
        in_specs=[
            pl.BlockSpec((1, 3, _NP), lambda i: (i, 0, 0)),
            pl.BlockSpec((1, _S, 3), lambda i: (i, 0, 0)),
        ],
        out_specs=[
            pl.BlockSpec((1, _S, _NP), lambda i: (i, 0, 0)),
            pl.BlockSpec((1, _S, 1), lambda i: (i, 0, 0)),
        ],
        out_shape=(
            jax.ShapeDtypeStruct((_B, _S, _NP), jnp.float32),
            jax.ShapeDtypeStruct((_B, _S, 1), jnp.float32),
        ),
    )(pos_t, q)
    return d2, tau.reshape(_B, _S)


_NQ = _B * _S
_NT = 32
_QPT = _NQ // _NT
_GD = 128


def _sc_mesh():
    return plsc.VectorSubcoreMesh(core_axis_name="c", subcore_axis_name="s")


def _sc_params():
    cp = pltpu.CompilerParams()
    if "needs_layout_passes" in pltpu.CompilerParams.__dataclass_fields__:
        cp = dataclasses.replace(cp, needs_layout_passes=False)
    return cp


def _compact_sc(d2_flat, tau_flat, selfg):
    @functools.partial(
        pl.kernel,
        mesh=_sc_mesh(),
        compiler_params=_sc_params(),
        out_type=jax.ShapeDtypeStruct((_NQ * _K,), jnp.int32),
        scratch_types=[
            pltpu.VMEM((_NP,), jnp.float32),
            pltpu.VMEM((_NP,), jnp.float32),
            pltpu.VMEM((_QPT + 16,), jnp.float32),
            pltpu.VMEM((_QPT + 16,), jnp.int32),
            pltpu.VMEM((128,), jnp.int32),
            pltpu.SemaphoreType.DMA,
            pltpu.SemaphoreType.DMA,
        ],
    )
    def k(d2_hbm, tau_hbm, selfg_hbm, nbr_hbm, rowA, rowB, tauv, selfv, outv,
          semA, semB):
        wid = lax.axis_index("s") * 2 + lax.axis_index("c")
        base = wid * _QPT
        pltpu.sync_copy(tau_hbm.at[pl.ds(base, _QPT)], tauv.at[pl.ds(0, _QPT)])
        pltpu.sync_copy(selfg_hbm.at[pl.ds(base, _QPT)],
                        selfv.at[pl.ds(0, _QPT)])

        def process(t, row):
            qglob = base + t
            tau = tauv[pl.ds(t, 16)][0]
            selfs = selfv[pl.ds(t, 16)][0]
            jbase = (qglob // _S) * _NP
            selfvec = lax.broadcast(selfs, (16,))
            for s in range(5):
                outv[pl.ds(16 * s, 16)] = selfvec
            tvec = lax.broadcast(tau, (16,))
            i16 = lax.broadcasted_iota(jnp.int32, (16,), 0)

            def chunk(c, off):
                v = row[pl.ds(c * 16, 16)]
                m = v <= tvec
                idx = i16 + (jbase + c * 16)
                r = plsc.cumsum(jnp.where(m, 1, 0))
                plsc.store_scatter(outv, [off + r - 1], idx, mask=m)
                return jnp.minimum(off + jnp.sum(jnp.where(m, 1, 0)), _K)

            lax.fori_loop(0, _NP // 16, chunk, jnp.int32(0))
            pltpu.sync_copy(outv.at[pl.ds(0, _K)],
                            nbr_hbm.at[pl.ds(qglob * _K, _K)])

        pltpu.make_async_copy(d2_hbm.at[base], rowA, semA).start()

        @pl.loop(0, _QPT, step=2)
        def _(t):
            pltpu.make_async_copy(d2_hbm.at[base + t + 1], rowB, semB).start()
            pltpu.make_async_copy(d2_hbm.at[base + t], rowA, semA).wait()
            process(t, rowA)

            @pl.when(t + 2 < _QPT)
            def _():
                pltpu.make_async_copy(d2_hbm.at[base + t + 2], rowA, semA).start()

            pltpu.make_async_copy(d2_hbm.at[base + t + 1], rowB, semB).wait()
            process(t + 1, rowB)

    return k(d2_flat, tau_flat, selfg)


def _gather_sc(xp, nbr_flat):
    tot = _NQ * _K
    ipt = tot // _NT
    ch = 256

    @functools.partial(
        pl.kernel,
        mesh=_sc_mesh(),
        out_type=jax.ShapeDtypeStruct((tot, _GD), jnp.float32),
        scratch_types=[
            pltpu.VMEM((ch,), jnp.int32),
            pltpu.VMEM((ch, _GD), jnp.float32),
            pltpu.SemaphoreType.DMA,
        ],
    )
    def k(xp_hbm, idx_hbm, o_hbm, idxv, rows, sem):
        wid = lax.axis_index("s") * 2 + lax.axis_index("c")
        base = wid * ipt

        @pl.loop(0, ipt, step=ch)
        def _(i):
            pltpu.sync_copy(idx_hbm.at[pl.ds(base + i, ch)], idxv)
            pltpu.async_copy(xp_hbm.at[idxv], rows, sem).wait()
            pltpu.sync_copy(rows, o_hbm.at[pl.ds(base + i, ch)])

    return k(xp, nbr_flat)


def _mlp_body(g_ref, q_ref, w1_ref, b1_ref, w2_ref, b2_ref, w3_ref, b3_ref,
              o_ref):
    g = g_ref[...]
    msg = jnp.concatenate([g[:, :_D], g[:, _D:_D + 3] - q_ref[...]], axis=1)
    h = jnp.maximum(
        jnp.dot(msg, w1_ref[...], preferred_element_type=jnp.float32) + b1_ref[...], 0.0)
    h = jnp.maximum(
        jnp.dot(h, w2_ref[...], preferred_element_type=jnp.float32) + b2_ref[...], 0.0)
    h = jnp.maximum(
        jnp.dot(h, w3_ref[...], preferred_element_type=jnp.float32) + b3_ref[...], 0.0)
    for _ in range(6):
        n = h.shape[0] // 2
        hr = h.reshape(n, 2, 128)
        h = jnp.maximum(hr[:, 0, :], hr[:, 1, :])
    o_ref[...] = h


def _mlp_max(g, qrep, W1, b1, W2, b2, W3, b3):
    npairs = g.shape[0]
    QB = 64
    rows = QB * _K
    grid = (npairs // rows,)
    return pl.pallas_call(
        _mlp_body,
        grid=grid,
        in_specs=[
            pl.BlockSpec((rows, _GD), lambda i: (i, 0)),
            pl.BlockSpec((rows, 3), lambda i: (i, 0)),
            pl.BlockSpec((67, 64), lambda i: (0, 0)),
            pl.BlockSpec((1, 64), lambda i: (0, 0)),
            pl.BlockSpec((64, 64), lambda i: (0, 0)),
            pl.BlockSpec((1, 64), lambda i: (0, 0)),
            pl.BlockSpec((64, 128), lambda i: (0, 0)),
            pl.BlockSpec((1, 128), lambda i: (0, 0)),
        ],
        out_specs=pl.BlockSpec((QB, 128), lambda i: (i, 0)),
        out_shape=jax.ShapeDtypeStruct((npairs // _K, 128), jnp.float32),
    )(g, qrep, W1, b1.reshape(1, 64), W2, b2.reshape(1, 64), W3,
      b3.reshape(1, 128))


def kernel(x, pos, batch, W1, b1, W2, b2, W3, b3):
    pos_b = pos.reshape(_B, _NP, 3)
    idx_local, q = _fps_all(pos_b)
    d2, tau = _ballq(pos_b, q)
    idx_global = (idx_local
                  + jnp.arange(_B, dtype=jnp.int32)[:, None] * _NP).reshape(-1)
    nbr = _compact_sc(d2.reshape(_NQ, _NP), tau.reshape(_NQ), idx_global)
    xp = jnp.concatenate(
        [x, pos, jnp.zeros((x.shape[0], _GD - _D - 3), jnp.float32)], axis=1)
    g = _gather_sc(xp, nbr)
    qrep = jnp.repeat(q.reshape(_NQ, 3), _K, axis=0)
    out = _mlp_max(g, qrep, W1, b1, W2, b2, W3, b3)
    x_out = out
    pos_out = q.reshape(_B * _S, 3)
    batch_out = batch[idx_global]
    return (x_out, pos_out, batch_out)


# speedup vs baseline: 16.4248x; 1.1190x over previous
import dataclasses
import functools

import jax
import jax.numpy as jnp
import numpy as np
from jax import lax
from jax.experimental import pallas as pl
from jax.experimental.pallas import tpu as pltpu
from jax.experimental.pallas import tpu_sc as plsc

_B = 16
_NP = 4096
_S = 512
_K = 64
_D = 64
_R = 0.2


_SL = 32


def _fps_body(px_ref, py_ref, pz_ref, idx_ref, qx_ref, qy_ref, qz_ref, d_ref):
    px = px_ref[...]
    py = py_ref[...]
    pz = pz_ref[...]
    li = (jax.lax.broadcasted_iota(jnp.int32, (1, _SL, 128), 1) * 128
          + jax.lax.broadcasted_iota(jnp.int32, (1, _SL, 128), 2))
    li_s = (jax.lax.broadcasted_iota(jnp.int32, (1, 4, 128), 1) * 128
            + jax.lax.broadcasted_iota(jnp.int32, (1, 4, 128), 2))

    def point_coords(nxt):
        sel = li == nxt
        sx = jnp.sum(jnp.where(sel, px, 0.0), axis=(1, 2), keepdims=True)
        sy = jnp.sum(jnp.where(sel, py, 0.0), axis=(1, 2), keepdims=True)
        sz = jnp.sum(jnp.where(sel, pz, 0.0), axis=(1, 2), keepdims=True)
        return sx, sy, sz

    zero = jnp.zeros((_B, 1, 1), jnp.int32)
    sx, sy, sz = point_coords(zero)
    d_ref[...] = (px - sx) ** 2 + (py - sy) ** 2 + (pz - sz) ** 2
    idx_ref[...] = jnp.zeros((_B, 4, 128), jnp.int32)
    qx_ref[...] = jnp.where(li_s == 0, sx, 0.0)
    qy_ref[...] = jnp.where(li_s == 0, sy, 0.0)
    qz_ref[...] = jnp.where(li_s == 0, sz, 0.0)

    def body(i, _):
        d = d_ref[...]
        m = jnp.max(d, axis=(1, 2), keepdims=True)
        nxt = jnp.min(jnp.where(d == m, li, jnp.int32(1 << 30)),
                      axis=(1, 2), keepdims=True)
        sx, sy, sz = point_coords(nxt)
        dn = (px - sx) ** 2 + (py - sy) ** 2 + (pz - sz) ** 2
        d_ref[...] = jnp.minimum(d, dn)
        at_i = li_s == i
        idx_ref[...] = jnp.where(at_i, nxt, idx_ref[...])
        qx_ref[...] = jnp.where(at_i, sx, qx_ref[...])
        qy_ref[...] = jnp.where(at_i, sy, qy_ref[...])
        qz_ref[...] = jnp.where(at_i, sz, qz_ref[...])
        return 0

    jax.lax.fori_loop(1, _S, body, 0)


def _fps_all(pos_b):
    px = pos_b[:, :, 0].reshape(_B, _SL, 128)
    py = pos_b[:, :, 1].reshape(_B, _SL, 128)
    pz = pos_b[:, :, 2].reshape(_B, _SL, 128)
    out_shapes = (
        jax.ShapeDtypeStruct((_B, 4, 128), jnp.int32),
        jax.ShapeDtypeStruct((_B, 4, 128), jnp.float32),
        jax.ShapeDtypeStruct((_B, 4, 128), jnp.float32),
        jax.ShapeDtypeStruct((_B, 4, 128), jnp.float32),
    )
    idxs, qx, qy, qz = pl.pallas_call(
        _fps_body,
        out_shape=out_shapes,
        scratch_shapes=[pltpu.VMEM((_B, _SL, 128), jnp.float32)],
    )(px, py, pz)
    idx_local = idxs.reshape(_B, _S)
    q = jnp.stack([qx.reshape(_B, _S), qy.reshape(_B, _S), qz.reshape(_B, _S)],
                  axis=-1)
    return idx_local, q


_R2F = np.float32(_R * _R)
_R2BITS = int(np.float32(_R * _R).view(np.int32))


def _ballq_body(pos_ref, q_ref, d2_ref, tau_ref):
    px = pos_ref[0, 0:1, :]
    py = pos_ref[0, 1:2, :]
    pz = pos_ref[0, 2:3, :]
    qx = q_ref[0, :, 0:1]
    qy = q_ref[0, :, 1:2]
    qz = q_ref[0, :, 2:3]
    d2 = (qx - px) ** 2 + (qy - py) ** 2 + (qz - pz) ** 2
    d2_ref[0] = d2
    d2b = jax.lax.bitcast_convert_type(d2, jnp.int32)
    in_r = d2 <= _R2F
    d2m = jnp.where(in_r, d2b, jnp.int32(0x7F7FFFFF))
    cnt0 = jnp.sum(in_r.astype(jnp.int32), axis=1, keepdims=True)
    g = jnp.minimum(cnt0, _K)

    def body(_, state):
        lo, hi = state
        mid = (lo + hi) >> 1
        cnt = jnp.sum((d2m <= mid).astype(jnp.int32), axis=1, keepdims=True)
        ge = cnt >= g
        return (jnp.where(ge, lo, mid + 1), jnp.where(ge, mid, hi))

    lo = jnp.zeros((_S, 1), jnp.int32)
    hi = jnp.full((_S, 1), jnp.int32(_R2BITS))
    lo, hi = jax.lax.fori_loop(0, 30, body, (lo, hi))
    tau_ref[0] = jax.lax.bitcast_convert_type(hi, jnp.float32)


def _ballq(pos_b, q):
    pos_t = pos_b.transpose(0, 2, 1)
    d2, tau = pl.pallas_call(
        _ballq_body,
        grid=(_B,),
        in_specs=[
            pl.BlockSpec((1, 3, _NP), lambda i: (i, 0, 0)),
            pl.BlockSpec((1, _S, 3), lambda i: (i, 0, 0)),
        ],
        out_specs=[
            pl.BlockSpec((1, _S, _NP), lambda i: (i, 0, 0)),
            pl.BlockSpec((1, _S, 1), lambda i: (i, 0, 0)),
        ],
        out_shape=(
            jax.ShapeDtypeStruct((_B, _S, _NP), jnp.float32),
            jax.ShapeDtypeStruct((_B, _S, 1), jnp.float32),
        ),
    )(pos_t, q)
    return d2, tau.reshape(_B, _S)


_NQ = _B * _S
_NT = 32
_QPT = _NQ // _NT
_GD = 128


def _sc_mesh():
    return plsc.VectorSubcoreMesh(core_axis_name="c", subcore_axis_name="s")


def _sc_params():
    cp = pltpu.CompilerParams()
    if "needs_layout_passes" in pltpu.CompilerParams.__dataclass_fields__:
        cp = dataclasses.replace(cp, needs_layout_passes=False)
    return cp


def _compact_sc(d2_flat, tau_flat, selfg):
    @functools.partial(
        pl.kernel,
        mesh=_sc_mesh(),
        compiler_params=_sc_params(),
        out_type=jax.ShapeDtypeStruct((_NQ * _K,), jnp.int32),
        scratch_types=[
            pltpu.VMEM((_NP,), jnp.float32),
            pltpu.VMEM((_NP,), jnp.float32),
            pltpu.VMEM((_QPT + 16,), jnp.float32),
            pltpu.VMEM((_QPT + 16,), jnp.int32),
            pltpu.VMEM((128,), jnp.int32),
            pltpu.SemaphoreType.DMA,
            pltpu.SemaphoreType.DMA,
        ],
    )
    def k(d2_hbm, tau_hbm, selfg_hbm, nbr_hbm, rowA, rowB, tauv, selfv, outv,
          semA, semB):
        wid = lax.axis_index("s") * 2 + lax.axis_index("c")
        base = wid * _QPT
        pltpu.sync_copy(tau_hbm.at[pl.ds(base, _QPT)], tauv.at[pl.ds(0, _QPT)])
        pltpu.sync_copy(selfg_hbm.at[pl.ds(base, _QPT)],
                        selfv.at[pl.ds(0, _QPT)])

        def process(t, row):
            qglob = base + t
            tau = tauv[pl.ds(t, 16)][0]
            selfs = selfv[pl.ds(t, 16)][0]
            jbase = (qglob // _S) * _NP
            selfvec = lax.broadcast(selfs, (16,))
            for s in range(5):
                outv[pl.ds(16 * s, 16)] = selfvec
            tvec = lax.broadcast(tau, (16,))
            i16 = lax.broadcasted_iota(jnp.int32, (16,), 0)

            def chunk(c, voff):
                v = row[pl.ds(c * 16, 16)]
                m = v <= tvec
                idx = i16 + (jbase + c * 16)
                plsc.store_compressed(outv.at[pl.ds(voff[0], 16)], idx, mask=m)
                pop = plsc.all_reduce_population_count(m)
                return jnp.minimum(voff + pop, _K)

            lax.fori_loop(0, _NP // 16, chunk,
                          jnp.zeros((16,), jnp.int32))
            pltpu.sync_copy(outv.at[pl.ds(0, _K)],
                            nbr_hbm.at[pl.ds(qglob * _K, _K)])

        pltpu.make_async_copy(d2_hbm.at[base], rowA, semA).start()

        @pl.loop(0, _QPT, step=2)
        def _(t):
            pltpu.make_async_copy(d2_hbm.at[base + t + 1], rowB, semB).start()
            pltpu.make_async_copy(d2_hbm.at[base + t], rowA, semA).wait()
            process(t, rowA)

            @pl.when(t + 2 < _QPT)
            def _():
                pltpu.make_async_copy(d2_hbm.at[base + t + 2], rowA, semA).start()

            pltpu.make_async_copy(d2_hbm.at[base + t + 1], rowB, semB).wait()
            process(t + 1, rowB)

    return k(d2_flat, tau_flat, selfg)


def _gather_sc(xp, nbr_flat):
    tot = _NQ * _K
    ipt = tot // _NT
    ch = 256

    @functools.partial(
        pl.kernel,
        mesh=_sc_mesh(),
        out_type=jax.ShapeDtypeStruct((tot, _GD), jnp.float32),
        scratch_types=[
            pltpu.VMEM((ch,), jnp.int32),
            pltpu.VMEM((ch, _GD), jnp.float32),
            pltpu.SemaphoreType.DMA,
        ],
    )
    def k(xp_hbm, idx_hbm, o_hbm, idxv, rows, sem):
        wid = lax.axis_index("s") * 2 + lax.axis_index("c")
        base = wid * ipt

        @pl.loop(0, ipt, step=ch)
        def _(i):
            pltpu.sync_copy(idx_hbm.at[pl.ds(base + i, ch)], idxv)
            pltpu.async_copy(xp_hbm.at[idxv], rows, sem).wait()
            pltpu.sync_copy(rows, o_hbm.at[pl.ds(base + i, ch)])

    return k(xp, nbr_flat)


def _mlp_body(g_ref, q_ref, w1_ref, b1_ref, w2_ref, b2_ref, w3_ref, b3_ref,
              o_ref):
    g = g_ref[...]
    msg = jnp.concatenate([g[:, :_D], g[:, _D:_D + 3] - q_ref[...]], axis=1)
    h = jnp.maximum(
        jnp.dot(msg, w1_ref[...], preferred_element_type=jnp.float32) + b1_ref[...], 0.0)
    h = jnp.maximum(
        jnp.dot(h, w2_ref[...], preferred_element_type=jnp.float32) + b2_ref[...], 0.0)
    h = jnp.maximum(
        jnp.dot(h, w3_ref[...], preferred_element_type=jnp.float32) + b3_ref[...], 0.0)
    for _ in range(6):
        n = h.shape[0] // 2
        hr = h.reshape(n, 2, 128)
        h = jnp.maximum(hr[:, 0, :], hr[:, 1, :])
    o_ref[...] = h


def _mlp_max(g, qrep, W1, b1, W2, b2, W3, b3):
    npairs = g.shape[0]
    QB = 64
    rows = QB * _K
    grid = (npairs // rows,)
    return pl.pallas_call(
        _mlp_body,
        grid=grid,
        in_specs=[
            pl.BlockSpec((rows, _GD), lambda i: (i, 0)),
            pl.BlockSpec((rows, 3), lambda i: (i, 0)),
            pl.BlockSpec((67, 64), lambda i: (0, 0)),
            pl.BlockSpec((1, 64), lambda i: (0, 0)),
            pl.BlockSpec((64, 64), lambda i: (0, 0)),
            pl.BlockSpec((1, 64), lambda i: (0, 0)),
            pl.BlockSpec((64, 128), lambda i: (0, 0)),
            pl.BlockSpec((1, 128), lambda i: (0, 0)),
        ],
        out_specs=pl.BlockSpec((QB, 128), lambda i: (i, 0)),
        out_shape=jax.ShapeDtypeStruct((npairs // _K, 128), jnp.float32),
    )(g, qrep, W1, b1.reshape(1, 64), W2, b2.reshape(1, 64), W3,
      b3.reshape(1, 128))


def kernel(x, pos, batch, W1, b1, W2, b2, W3, b3):
    pos_b = pos.reshape(_B, _NP, 3)
    idx_local, q = _fps_all(pos_b)
    d2, tau = _ballq(pos_b, q)
    idx_global = (idx_local
                  + jnp.arange(_B, dtype=jnp.int32)[:, None] * _NP).reshape(-1)
    nbr = _compact_sc(d2.reshape(_NQ, _NP), tau.reshape(_NQ), idx_global)
    xp = jnp.concatenate(
        [x, pos, jnp.zeros((x.shape[0], _GD - _D - 3), jnp.float32)], axis=1)
    g = _gather_sc(xp, nbr)
    qrep = jnp.repeat(q.reshape(_NQ, 3), _K, axis=0)
    out = _mlp_max(g, qrep, W1, b1, W2, b2, W3, b3)
    x_out = out
    pos_out = q.reshape(_B * _S, 3)
    batch_out = batch[idx_global]
    return (x_out, pos_out, batch_out)
